# Initial kernel scaffold; baseline (speedup 1.0000x reference)
import jax
import jax.numpy as jnp
from jax.experimental import pallas as pl


def kernel(updates, mask):
    raise NotImplementedError("write your pallas kernel here")


# speedup vs baseline: 40.9030x; 40.9030x over previous
import dataclasses
import functools

import jax
import jax.numpy as jnp
from jax import lax
from jax.experimental import pallas as pl
from jax.experimental.pallas import tpu as pltpu
from jax.experimental.pallas import tpu_sc as plsc

_B, _H, _W, _C = 4, 112, 112, 192
_HW = _H * _W
_S = (2 * _H) * (2 * _W)
_NCH = _B * _C
_NW = 32
_CPW = _NCH // _NW


def _compiler_params():
    cp = pltpu.CompilerParams()
    fields = pltpu.CompilerParams.__dataclass_fields__
    if "needs_layout_passes" in fields:
        cp = dataclasses.replace(cp, needs_layout_passes=False)
    if "use_tc_tiling_on_sc" in fields:
        cp = dataclasses.replace(cp, use_tc_tiling_on_sc=True)
    return cp


def _sc_unpool(upd_t, msk_t):
    mesh = plsc.VectorSubcoreMesh(core_axis_name="c", subcore_axis_name="s")

    @functools.partial(
        pl.kernel,
        out_type=jax.ShapeDtypeStruct((_B, _C, _S), jnp.float32),
        mesh=mesh,
        compiler_params=_compiler_params(),
        scratch_types=[
            pltpu.VMEM((_HW,), jnp.float32),
            pltpu.VMEM((_HW,), jnp.int32),
            pltpu.VMEM((_S,), jnp.float32),
            pltpu.VMEM((_S,), jnp.float32),
            pltpu.SemaphoreType.DMA,
            pltpu.SemaphoreType.DMA,
            pltpu.SemaphoreType.DMA,
        ],
    )
    def k(upd_hbm, msk_hbm, out_hbm, u_v, m_v, acc_a, acc_b,
          sem_in, sem_oa, sem_ob):
        cid = lax.axis_index("c")
        sid = lax.axis_index("s")
        wid = sid * 2 + cid
        bi = lax.shift_right_logical(wid, 3)
        base = lax.bitwise_and(wid, 7) * _CPW
        third = jnp.float32(1.0 / 3.0)
        zvec = jnp.zeros((16,), jnp.float32)

        def start_in(ci):
            pltpu.make_async_copy(upd_hbm.at[bi, ci], u_v, sem_in).start()
            pltpu.make_async_copy(msk_hbm.at[bi, ci], m_v, sem_in).start()

        def wait_in(ci):
            pltpu.make_async_copy(upd_hbm.at[bi, ci], u_v, sem_in).wait()
            pltpu.make_async_copy(msk_hbm.at[bi, ci], m_v, sem_in).wait()

        def zero(acc):
            @pl.loop(0, _S, step=64)
            def _(i):
                acc[pl.ds(i, 16)] = zvec
                acc[pl.ds(i + 16, 16)] = zvec
                acc[pl.ds(i + 32, 16)] = zvec
                acc[pl.ds(i + 48, 16)] = zvec

        def scatter(acc):
            @pl.loop(0, _HW, step=64)
            def _(i):
                for o in range(0, 64, 16):
                    m = m_v[pl.ds(i + o, 16)]
                    u = u_v[pl.ds(i + o, 16)]
                    n6 = lax.shift_right_logical(m, 6)
                    p = (n6.astype(jnp.float32) * third).astype(jnp.int32)
                    plsc.addupdate_scatter(acc, [p], u)

        start_in(base)

        @pl.loop(0, _CPW // 2)
        def _(t):
            ca = base + 2 * t
            cb = ca + 1

            @pl.when(t > 0)
            def _():
                pltpu.make_async_copy(acc_a, out_hbm.at[bi, ca - 2], sem_oa).wait()

            zero(acc_a)
            wait_in(ca)
            scatter(acc_a)
            start_in(cb)
            pltpu.make_async_copy(acc_a, out_hbm.at[bi, ca], sem_oa).start()

            @pl.when(t > 0)
            def _():
                pltpu.make_async_copy(acc_b, out_hbm.at[bi, cb - 2], sem_ob).wait()

            zero(acc_b)
            wait_in(cb)
            scatter(acc_b)

            @pl.when(t < _CPW // 2 - 1)
            def _():
                start_in(cb + 1)

            pltpu.make_async_copy(acc_b, out_hbm.at[bi, cb], sem_ob).start()

        pltpu.make_async_copy(acc_a, out_hbm.at[bi, base + _CPW - 2], sem_oa).wait()
        pltpu.make_async_copy(acc_b, out_hbm.at[bi, base + _CPW - 1], sem_ob).wait()

    return k(upd_t, msk_t)


def kernel(updates, mask):
    m32 = mask.astype(jnp.int32)
    upd_t = updates.reshape(_B, _HW, _C).transpose(0, 2, 1)
    msk_t = m32.reshape(_B, _HW, _C).transpose(0, 2, 1)
    out_t = _sc_unpool(upd_t, msk_t)
    return out_t.transpose(0, 2, 1).reshape(_B, 2 * _H, 2 * _W, _C)
